# baseline (speedup 1.0000x reference)
import functools

import jax
import jax.numpy as jnp
from jax import lax
from jax.experimental import pallas as pl
from jax.experimental.pallas import tpu as pltpu
from jax.experimental.pallas import tpu_sc as plsc

BS = 8
HQ, WQ = 32, 32
HC, WC = 64, 64
LQ = HQ * WQ
LC = HC * WC
QD = 768
CD = 768
HEADS = 12
DH = 64
INNER = HEADS * DH
NPTS = 4
NSLOT = NPTS * 4

NCORES = 2
NSUB = 16
NWORK = NCORES * NSUB
NQTR = 4
UNITS = BS * HEADS * NQTR
UNITS_PER_W = UNITS // NWORK
QBLK = 16
NBLK = LQ // QBLK
CTX_TILE = 512


def _fuse_body(wq_ref, wcat_ref, wk_ref, wval_ref, wproj_ref, wout_ref,
               bproj_ref, bout_ref, wqc_ref, wkv_ref, wpo_ref, bf_ref):
    wqc_ref[...] = lax.dot_general(wcat_ref[...], wq_ref[...],
                                   (((0,), (1,)), ((), ())),
                                   preferred_element_type=jnp.float32)
    wkv_ref[...] = jnp.dot(wk_ref[...], wval_ref[...],
                           preferred_element_type=jnp.float32)
    wpo = jnp.dot(wproj_ref[...], wout_ref[...],
                  preferred_element_type=jnp.float32)
    wpo_ref[...] = wpo
    bf_ref[...] = jnp.dot(bproj_ref[...], wpo,
                          preferred_element_type=jnp.float32) + bout_ref[...]


def _fuse_weights(wq, wcat, wk, wval, wproj, wout, bproj, bout):
    return pl.pallas_call(
        _fuse_body,
        out_shape=[
            jax.ShapeDtypeStruct((144, QD), jnp.float32),
            jax.ShapeDtypeStruct((CD, INNER), jnp.float32),
            jax.ShapeDtypeStruct((INNER, QD), jnp.float32),
            jax.ShapeDtypeStruct((1, QD), jnp.float32),
        ],
    )(wq, wcat, wk, wval, wproj, wout, bproj, bout)


def _value_body(ctx_ref, wkv_ref, bval_ref, out_ref):
    vt = lax.dot_general(wkv_ref[...], ctx_ref[0],
                         (((0,), (1,)), ((), ())),
                         preferred_element_type=jnp.float32)
    out_ref[0] = (vt + bval_ref[...]).reshape(HEADS * NQTR, 16, CTX_TILE)


def _value_proj(context, wkv, bval):
    nb = context.shape[0]
    return pl.pallas_call(
        _value_body,
        grid=(nb, LC // CTX_TILE),
        in_specs=[
            pl.BlockSpec((1, CTX_TILE, CD), lambda b, i: (b, i, 0)),
            pl.BlockSpec((CD, INNER), lambda b, i: (0, 0)),
            pl.BlockSpec((INNER, 1), lambda b, i: (0, 0)),
        ],
        out_specs=pl.BlockSpec((1, HEADS * NQTR, 16, CTX_TILE),
                               lambda b, i: (b, 0, 0, i)),
        out_shape=jax.ShapeDtypeStruct((nb, HEADS * NQTR, 16, LC),
                                       jnp.float32),
    )(context, wkv, bval)


def _samp_body(x_ref, wqc_ref, boffc_ref, g_ref, w4_ref, i4_ref):
    xb = x_ref[0]
    t = lax.dot_general(wqc_ref[...], xb, (((1,), (1,)), ((), ())),
                        preferred_element_type=jnp.float32) + boffc_ref[...]
    offx = t[0:48, :]
    offy = t[48:96, :]
    att = t[96:144, :]
    e = jnp.exp(att)
    aw = e / jnp.dot(g_ref[...], e, preferred_element_type=jnp.float32)

    qi = lax.broadcasted_iota(jnp.int32, (48, LQ), 1)
    qxf = (qi % WQ).astype(jnp.float32)
    qyf = (qi // WQ).astype(jnp.float32)
    ax = 2.0 * qxf + 0.5 + offx
    ay = 2.0 * qyf + 0.5 + offy
    x0 = jnp.floor(ax)
    y0 = jnp.floor(ay)
    fx = ax - x0
    fy = ay - y0
    x1 = x0 + 1.0
    y1 = y0 + 1.0

    def corner(xc, yc, wx, wy):
        inb = ((xc >= 0.0) & (xc <= WC - 1.0)
               & (yc >= 0.0) & (yc <= HC - 1.0)).astype(jnp.float32)
        xcc = jnp.clip(xc, 0.0, WC - 1.0)
        ycc = jnp.clip(yc, 0.0, HC - 1.0)
        idx = (ycc * WC + xcc).astype(jnp.int32)
        return aw * wx * wy * inb, idx

    w0, i0 = corner(x0, y0, 1.0 - fx, 1.0 - fy)
    w1, i1 = corner(x1, y0, fx, 1.0 - fy)
    w2, i2 = corner(x0, y1, 1.0 - fx, fy)
    w3, i3 = corner(x1, y1, fx, fy)
    w4_ref[0, :, 0, :] = w0
    w4_ref[0, :, 1, :] = w1
    w4_ref[0, :, 2, :] = w2
    w4_ref[0, :, 3, :] = w3
    i4_ref[0, :, 0, :] = i0
    i4_ref[0, :, 1, :] = i1
    i4_ref[0, :, 2, :] = i2
    i4_ref[0, :, 3, :] = i3


def _samp_params(x, wqc, boffc, g):
    nb = x.shape[0]
    return pl.pallas_call(
        _samp_body,
        grid=(nb,),
        in_specs=[
            pl.BlockSpec((1, LQ, QD), lambda b: (b, 0, 0)),
            pl.BlockSpec((144, QD), lambda b: (0, 0)),
            pl.BlockSpec((144, 1), lambda b: (0, 0)),
            pl.BlockSpec((48, 48), lambda b: (0, 0)),
        ],
        out_specs=[
            pl.BlockSpec((1, 48, 4, LQ), lambda b: (b, 0, 0, 0)),
            pl.BlockSpec((1, 48, 4, LQ), lambda b: (b, 0, 0, 0)),
        ],
        out_shape=[
            jax.ShapeDtypeStruct((nb, 48, 4, LQ), jnp.float32),
            jax.ShapeDtypeStruct((nb, 48, 4, LQ), jnp.int32),
        ],
    )(x, wqc, boffc, g)


def _make_sc_body(units_per_w):
    def _sc_body(val_hbm, idx_hbm, w_hbm, out_hbm, table_v, idx_v, w_v,
                 out_v):
        wid = lax.axis_index("s") * NCORES + lax.axis_index("c")

        def unit_body(u, carry):
            g = u * NWORK + wid
            b = g // (HEADS * NQTR)
            r = g % (HEADS * NQTR)
            h = r // NQTR
            pltpu.sync_copy(val_hbm.at[b, r], table_v)
            pltpu.sync_copy(idx_hbm.at[b, h], idx_v)
            pltpu.sync_copy(w_hbm.at[b, h], w_v)

            @plsc.parallel_loop(0, NBLK, 1, unroll=4)
            def blk_body(blk):
                qbase = blk * QBLK
                bvecs = [idx_v[pl.ds(s * LQ + qbase, QBLK)]
                         for s in range(NSLOT)]
                wvecs = [w_v[pl.ds(s * LQ + qbase, QBLK)]
                         for s in range(NSLOT)]
                for ch in range(16):
                    row = jnp.full((QBLK,), ch, dtype=jnp.int32)
                    vals = [plsc.load_gather(table_v, [row, bvecs[s]])
                            for s in range(NSLOT)]
                    accs = [wvecs[0] * vals[0], wvecs[1] * vals[1],
                            wvecs[2] * vals[2], wvecs[3] * vals[3]]
                    for s in range(4, NSLOT):
                        accs[s % 4] = accs[s % 4] + wvecs[s] * vals[s]
                    out_v[pl.ds(ch * LQ + blk * QBLK, QBLK)] = (
                        (accs[0] + accs[1]) + (accs[2] + accs[3]))
            pltpu.sync_copy(out_v, out_hbm.at[b, r])
            return carry

        lax.fori_loop(0, units_per_w, unit_body, 0)

    return _sc_body


def _sc_sample(value4, islots, wslots):
    nb = value4.shape[0]
    units_per_w = nb * HEADS * NQTR // NWORK
    return pl.kernel(
        _make_sc_body(units_per_w),
        out_type=jax.ShapeDtypeStruct((nb, HEADS * NQTR, 16 * LQ),
                                      jnp.float32),
        mesh=plsc.VectorSubcoreMesh(core_axis_name="c", subcore_axis_name="s"),
        compiler_params=pltpu.CompilerParams(needs_layout_passes=False),
        scratch_types=[
            pltpu.VMEM((16, LC), jnp.float32),
            pltpu.VMEM((NBLK * NSLOT * QBLK,), jnp.int32),
            pltpu.VMEM((NBLK * NSLOT * QBLK,), jnp.float32),
            pltpu.VMEM((16 * LQ,), jnp.float32),
        ],
    )(value4, islots, wslots)


def _out_body(s_ref, wpo_ref, bf_ref, y_ref):
    sm = s_ref[0].reshape(INNER, LQ)
    y_ref[0] = lax.dot_general(
        sm, wpo_ref[...], (((0,), (0,)), ((), ())),
        preferred_element_type=jnp.float32) + bf_ref[...]


def _out_proj(s, wpo, bf):
    nb = s.shape[0]
    return pl.pallas_call(
        _out_body,
        grid=(nb,),
        in_specs=[
            pl.BlockSpec((1, HEADS * NQTR, 16 * LQ), lambda b: (b, 0, 0)),
            pl.BlockSpec((INNER, QD), lambda b: (0, 0)),
            pl.BlockSpec((1, QD), lambda b: (0, 0)),
        ],
        out_specs=pl.BlockSpec((1, LQ, QD), lambda b: (b, 0, 0)),
        out_shape=jax.ShapeDtypeStruct((nb, LQ, QD), jnp.float32),
    )(s, wpo, bf)


def kernel(x, context, spatial_shapes, spatial_shapes_c, W_q, W_k, W_off,
           b_off, W_attn, b_attn, W_val, b_val, W_proj, b_proj, W_out, b_out):
    wcat = jnp.concatenate([W_off[:, 0::2], W_off[:, 1::2], W_attn], axis=1)
    boffc = jnp.concatenate([b_off[0::2], b_off[1::2], b_attn])[:, None]
    lane = jnp.arange(48) // NPTS
    g = (lane[:, None] == lane[None, :]).astype(jnp.float32)

    wqc, wkv, wpo, bf = _fuse_weights(
        W_q, wcat, W_k, W_val, W_proj, W_out, b_proj[None, :], b_out[None, :])

    value4 = _value_proj(context, wkv, b_val[:, None])
    w4, i4 = _samp_params(x, wqc, boffc, g)
    wslots = w4.reshape(BS, HEADS, NSLOT * LQ)
    islots = i4.reshape(BS, HEADS, NSLOT * LQ)
    s = _sc_sample(value4, islots, wslots)
    return _out_proj(s, wpo, bf)


# speedup vs baseline: 1.6613x; 1.6613x over previous
import functools

import jax
import jax.numpy as jnp
from jax import lax
from jax.experimental import pallas as pl
from jax.experimental.pallas import tpu as pltpu
from jax.experimental.pallas import tpu_sc as plsc

BS = 8
HQ, WQ = 32, 32
HC, WC = 64, 64
LQ = HQ * WQ
LC = HC * WC
QD = 768
CD = 768
HEADS = 12
DH = 64
INNER = HEADS * DH
NPTS = 4
NSLOT = NPTS * 4

NCORES = 2
NSUB = 16
NWORK = NCORES * NSUB
NQTR = 4
UNITS = BS * HEADS * NQTR
UNITS_PER_W = UNITS // NWORK
QBLK = 16
NBLK = LQ // QBLK
CTX_TILE = 512


def _fuse_body(wq_ref, wcat_ref, wk_ref, wval_ref, wproj_ref, wout_ref,
               bproj_ref, bout_ref, wqc_ref, wkv_ref, wpo_ref, bf_ref):
    wqc_ref[...] = lax.dot_general(wcat_ref[...], wq_ref[...],
                                   (((0,), (1,)), ((), ())),
                                   preferred_element_type=jnp.float32)
    wkv_ref[...] = jnp.dot(wk_ref[...], wval_ref[...],
                           preferred_element_type=jnp.float32)
    wpo = jnp.dot(wproj_ref[...], wout_ref[...],
                  preferred_element_type=jnp.float32)
    wpo_ref[...] = wpo
    bf_ref[...] = jnp.dot(bproj_ref[...], wpo,
                          preferred_element_type=jnp.float32) + bout_ref[...]


def _fuse_weights(wq, wcat, wk, wval, wproj, wout, bproj, bout):
    return pl.pallas_call(
        _fuse_body,
        out_shape=[
            jax.ShapeDtypeStruct((144, QD), jnp.float32),
            jax.ShapeDtypeStruct((CD, INNER), jnp.float32),
            jax.ShapeDtypeStruct((INNER, QD), jnp.float32),
            jax.ShapeDtypeStruct((1, QD), jnp.float32),
        ],
    )(wq, wcat, wk, wval, wproj, wout, bproj, bout)


def _value_body(ctx_ref, wkv_ref, bval_ref, out_ref):
    vt = lax.dot_general(wkv_ref[...], ctx_ref[0],
                         (((0,), (1,)), ((), ())),
                         preferred_element_type=jnp.float32)
    out_ref[0] = (vt + bval_ref[...]).reshape(HEADS * NQTR, 16, CTX_TILE)


def _value_proj(context, wkv, bval):
    nb = context.shape[0]
    return pl.pallas_call(
        _value_body,
        grid=(nb, LC // CTX_TILE),
        in_specs=[
            pl.BlockSpec((1, CTX_TILE, CD), lambda b, i: (b, i, 0)),
            pl.BlockSpec((CD, INNER), lambda b, i: (0, 0)),
            pl.BlockSpec((INNER, 1), lambda b, i: (0, 0)),
        ],
        out_specs=pl.BlockSpec((1, HEADS * NQTR, 16, CTX_TILE),
                               lambda b, i: (b, 0, 0, i)),
        out_shape=jax.ShapeDtypeStruct((nb, HEADS * NQTR, 16, LC),
                                       jnp.float32),
    )(context, wkv, bval)


def _samp_body(x_ref, wqc_ref, boffc_ref, g_ref, w4_ref, i4_ref):
    xb = x_ref[0]
    t = lax.dot_general(wqc_ref[...], xb, (((1,), (1,)), ((), ())),
                        preferred_element_type=jnp.float32) + boffc_ref[...]
    offx = t[0:48, :]
    offy = t[48:96, :]
    att = t[96:144, :]
    e = jnp.exp(att)
    aw = e / jnp.dot(g_ref[...], e, preferred_element_type=jnp.float32)

    qi = lax.broadcasted_iota(jnp.int32, (48, LQ), 1)
    qxf = (qi % WQ).astype(jnp.float32)
    qyf = (qi // WQ).astype(jnp.float32)
    ax = 2.0 * qxf + 0.5 + offx
    ay = 2.0 * qyf + 0.5 + offy
    x0 = jnp.floor(ax)
    y0 = jnp.floor(ay)
    fx = ax - x0
    fy = ay - y0
    x1 = x0 + 1.0
    y1 = y0 + 1.0

    def corner(xc, yc, wx, wy):
        inb = ((xc >= 0.0) & (xc <= WC - 1.0)
               & (yc >= 0.0) & (yc <= HC - 1.0)).astype(jnp.float32)
        xcc = jnp.clip(xc, 0.0, WC - 1.0)
        ycc = jnp.clip(yc, 0.0, HC - 1.0)
        idx = (ycc * WC + xcc).astype(jnp.int32)
        return aw * wx * wy * inb, idx

    w0, i0 = corner(x0, y0, 1.0 - fx, 1.0 - fy)
    w1, i1 = corner(x1, y0, fx, 1.0 - fy)
    w2, i2 = corner(x0, y1, 1.0 - fx, fy)
    w3, i3 = corner(x1, y1, fx, fy)
    w4_ref[0, :, 0, :] = w0
    w4_ref[0, :, 1, :] = w1
    w4_ref[0, :, 2, :] = w2
    w4_ref[0, :, 3, :] = w3
    i4_ref[0, :, 0, :] = i0
    i4_ref[0, :, 1, :] = i1
    i4_ref[0, :, 2, :] = i2
    i4_ref[0, :, 3, :] = i3


def _samp_params(x, wqc, boffc, g):
    nb = x.shape[0]
    return pl.pallas_call(
        _samp_body,
        grid=(nb,),
        in_specs=[
            pl.BlockSpec((1, LQ, QD), lambda b: (b, 0, 0)),
            pl.BlockSpec((144, QD), lambda b: (0, 0)),
            pl.BlockSpec((144, 1), lambda b: (0, 0)),
            pl.BlockSpec((48, 48), lambda b: (0, 0)),
        ],
        out_specs=[
            pl.BlockSpec((1, 48, 4, LQ), lambda b: (b, 0, 0, 0)),
            pl.BlockSpec((1, 48, 4, LQ), lambda b: (b, 0, 0, 0)),
        ],
        out_shape=[
            jax.ShapeDtypeStruct((nb, 48, 4, LQ), jnp.float32),
            jax.ShapeDtypeStruct((nb, 48, 4, LQ), jnp.int32),
        ],
    )(x, wqc, boffc, g)


def _make_sc_body(units_per_w):
    def _sc_body(val_hbm, idx_hbm, w_hbm, out_hbm, table_v, idx_v, w_v,
                 out_v):
        wid = lax.axis_index("s") * NCORES + lax.axis_index("c")

        def unit_body(u, carry):
            g = u * NWORK + wid
            b = g // (HEADS * NQTR)
            r = g % (HEADS * NQTR)
            h = r // NQTR
            pltpu.sync_copy(val_hbm.at[b, r], table_v)
            pltpu.sync_copy(idx_hbm.at[b, h], idx_v)
            pltpu.sync_copy(w_hbm.at[b, h], w_v)

            @plsc.parallel_loop(0, NBLK, 1, unroll=1)
            def blk_body(blk):
                qbase = blk * QBLK
                bvecs = [idx_v[pl.ds(s * LQ + qbase, QBLK)]
                         for s in range(NSLOT)]
                wvecs = [w_v[pl.ds(s * LQ + qbase, QBLK)]
                         for s in range(NSLOT)]
                for ch in range(16):
                    row = jnp.full((QBLK,), ch, dtype=jnp.int32)
                    vals = [plsc.load_gather(table_v, [row, bvecs[s]])
                            for s in range(NSLOT)]
                    accs = [wvecs[0] * vals[0], wvecs[1] * vals[1],
                            wvecs[2] * vals[2], wvecs[3] * vals[3]]
                    for s in range(4, NSLOT):
                        accs[s % 4] = accs[s % 4] + wvecs[s] * vals[s]
                    out_v[pl.ds(ch * LQ + blk * QBLK, QBLK)] = (
                        (accs[0] + accs[1]) + (accs[2] + accs[3]))
            pltpu.sync_copy(out_v, out_hbm.at[b, r])
            return carry

        lax.fori_loop(0, units_per_w, unit_body, 0)

    return _sc_body


def _sc_sample(value4, islots, wslots):
    nb = value4.shape[0]
    units_per_w = nb * HEADS * NQTR // NWORK
    return pl.kernel(
        _make_sc_body(units_per_w),
        out_type=jax.ShapeDtypeStruct((nb, HEADS * NQTR, 16 * LQ),
                                      jnp.float32),
        mesh=plsc.VectorSubcoreMesh(core_axis_name="c", subcore_axis_name="s"),
        compiler_params=pltpu.CompilerParams(needs_layout_passes=False),
        scratch_types=[
            pltpu.VMEM((16, LC), jnp.float32),
            pltpu.VMEM((NBLK * NSLOT * QBLK,), jnp.int32),
            pltpu.VMEM((NBLK * NSLOT * QBLK,), jnp.float32),
            pltpu.VMEM((16 * LQ,), jnp.float32),
        ],
    )(value4, islots, wslots)


def _out_body(s_ref, wpo_ref, bf_ref, y_ref):
    sm = s_ref[0].reshape(INNER, LQ)
    y_ref[0] = lax.dot_general(
        sm, wpo_ref[...], (((0,), (0,)), ((), ())),
        preferred_element_type=jnp.float32) + bf_ref[...]


def _out_proj(s, wpo, bf):
    nb = s.shape[0]
    return pl.pallas_call(
        _out_body,
        grid=(nb,),
        in_specs=[
            pl.BlockSpec((1, HEADS * NQTR, 16 * LQ), lambda b: (b, 0, 0)),
            pl.BlockSpec((INNER, QD), lambda b: (0, 0)),
            pl.BlockSpec((1, QD), lambda b: (0, 0)),
        ],
        out_specs=pl.BlockSpec((1, LQ, QD), lambda b: (b, 0, 0)),
        out_shape=jax.ShapeDtypeStruct((nb, LQ, QD), jnp.float32),
    )(s, wpo, bf)


def kernel(x, context, spatial_shapes, spatial_shapes_c, W_q, W_k, W_off,
           b_off, W_attn, b_attn, W_val, b_val, W_proj, b_proj, W_out, b_out):
    wcat = jnp.concatenate([W_off[:, 0::2], W_off[:, 1::2], W_attn], axis=1)
    boffc = jnp.concatenate([b_off[0::2], b_off[1::2], b_attn])[:, None]
    lane = jnp.arange(48) // NPTS
    g = (lane[:, None] == lane[None, :]).astype(jnp.float32)

    wqc, wkv, wpo, bf = _fuse_weights(
        W_q, wcat, W_k, W_val, W_proj, W_out, b_proj[None, :], b_out[None, :])

    value4 = _value_proj(context, wkv, b_val[:, None])
    w4, i4 = _samp_params(x, wqc, boffc, g)
    wslots = w4.reshape(BS, HEADS, NSLOT * LQ)
    islots = i4.reshape(BS, HEADS, NSLOT * LQ)
    s = _sc_sample(value4, islots, wslots)
    return _out_proj(s, wpo, bf)
